# baseline (speedup 1.0000x reference)
import jax
import jax.numpy as jnp
from jax import lax
from jax.experimental import pallas as pl
from jax.experimental.pallas import tpu as pltpu
from jax.experimental.pallas import tpu_sc as plsc

_NCLASS = 1000
_D = 512
_B = 16384
_NC = 2
_NS = 16
_NT = _NC * _NS
_L = 16
_CPT = 32
_G = 16
_NVEC = _D // _L
_RB = 2048


def _sc_body(feat_hbm, lab_hbm, out_hbm,
             labs_v, rowbuf, labbuf, rows_v, rows_w, tab_v, cnt_v,
             out_v, sem0, sem1):
    c = lax.axis_index("c")
    s = lax.axis_index("s")
    t = s * _NC + c
    zvec = jnp.zeros((_L,), jnp.float32)
    ovec = jnp.ones((_L,), jnp.float32)
    iota = lax.iota(jnp.int32, _L)
    five = jnp.full((_L,), 5, jnp.int32)
    onei = jnp.full((_L,), 1, jnp.int32)
    tvec = jnp.broadcast_to(t, (_L,))
    cvec0 = jnp.broadcast_to(t * _CPT, (_L,))
    dumpvec = jnp.broadcast_to(t * _CPT + _CPT, (_L,))

    pltpu.sync_copy(lab_hbm, labs_v)

    def zrow(i, _):
        for cc in range(_NVEC):
            tab_v[i, pl.ds(cc * _L, _L)] = zvec
        cnt_v[i] = 0.0
        return 0

    lax.fori_loop(0, _CPT + 1, zrow, 0)

    sixteen = jnp.full((_L,), _L, jnp.int32)

    def scan_body(i, offv):
        lv0 = labs_v[pl.ds(i * 2 * _L, _L)]
        lv1 = labs_v[pl.ds(i * 2 * _L + _L, _L)]
        m0 = lax.shift_right_logical(lv0, five) == tvec
        m1 = lax.shift_right_logical(lv1, five) == tvec
        rid0 = iota + jnp.broadcast_to(i * 2 * _L, (_L,))
        rid1 = rid0 + sixteen
        cs0 = plsc.cumsum(jnp.where(m0, onei, onei - onei))
        cs1 = plsc.cumsum(jnp.where(m1, onei, onei - onei))
        pos0 = (cs0 - onei) + offv
        offv1 = offv + plsc.all_reduce_population_count(m0)
        pos1 = (cs1 - onei) + offv1
        plsc.store_scatter(rowbuf, [pos0], rid0, mask=m0)
        plsc.store_scatter(labbuf, [pos0], lv0, mask=m0)
        plsc.store_scatter(rowbuf, [pos1], rid1, mask=m1)
        plsc.store_scatter(labbuf, [pos1], lv1, mask=m1)
        return offv1 + plsc.all_reduce_population_count(m1)

    offv = lax.fori_loop(0, _B // (2 * _L), scan_body,
                         jnp.zeros((_L,), jnp.int32))
    n = offv[0]
    zveci = jnp.zeros((_L,), jnp.int32)
    rowbuf[pl.ds(n, _L)] = zveci
    rowbuf[pl.ds(n + _L, _L)] = zveci
    labbuf[pl.ds(n, _L)] = dumpvec
    labbuf[pl.ds(n + _L, _L)] = dumpvec

    npair = (n + 2 * _G - 1) // (2 * _G)
    ntot2 = npair * 2

    def start(cid, buf, sem):
        pltpu.async_copy(feat_hbm.at[rowbuf.at[pl.ds(cid * _G, _G)]], buf, sem)

    def wait(cid, buf, sem):
        pltpu.make_async_copy(
            feat_hbm.at[rowbuf.at[pl.ds(cid * _G, _G)]], buf, sem).wait()

    @pl.when(ntot2 > 0)
    def _p0():
        start(0, rows_v, sem0)

    @pl.when(ntot2 > 1)
    def _p1():
        start(1, rows_w, sem1)

    def acc_chunk(cid, buf):
        lvec = labbuf[pl.ds(cid * _G, _L)] - cvec0
        for r in range(_G):
            lc = lvec[r]
            cnt_v[lc] = cnt_v[lc] + 1.0

            @plsc.parallel_loop(0, _NVEC, unroll=16)
            def _cc(cc):
                v = buf[r, pl.ds(cc * _L, _L)]
                plsc.addupdate(tab_v.at[lc, pl.ds(cc * _L, _L)], v)

    def pair(pid, _):
        cid0 = 2 * pid
        wait(cid0, rows_v, sem0)
        acc_chunk(cid0, rows_v)

        @pl.when(cid0 + 2 < ntot2)
        def _n0():
            start(cid0 + 2, rows_v, sem0)

        cid1 = cid0 + 1
        wait(cid1, rows_w, sem1)
        acc_chunk(cid1, rows_w)

        @pl.when(cid1 + 2 < ntot2)
        def _n1():
            start(cid1 + 2, rows_w, sem1)

        return 0

    lax.fori_loop(0, npair, pair, 0)

    def crow(lc, ctr):
        cvec = jnp.broadcast_to(cnt_v[lc], (_L,))
        inv = ovec / jnp.maximum(cvec, ovec)
        rowacc = zvec
        for cc in range(_NVEC):
            v = tab_v[lc, pl.ds(cc * _L, _L)]
            rowacc = rowacc + v * v
        return ctr + rowacc * inv

    ctr = lax.fori_loop(0, _CPT, crow, zvec)
    out_v[pl.ds(0, _L)] = ctr
    pltpu.sync_copy(out_v, out_hbm.at[t])


def _ssq_body(x_ref, o_ref):
    i = pl.program_id(0)

    @pl.when(i == 0)
    def _init():
        o_ref[...] = jnp.zeros_like(o_ref)

    x = x_ref[...]
    o_ref[...] = o_ref[...] + jnp.sum(x * x)


@jax.jit
def _center_loss_sc(features, labels):
    ctr_parts = pl.kernel(
        _sc_body,
        out_type=jax.ShapeDtypeStruct((_NT, _L), jnp.float32),
        mesh=plsc.VectorSubcoreMesh(core_axis_name="c", subcore_axis_name="s"),
        compiler_params=pltpu.CompilerParams(needs_layout_passes=False),
        cost_estimate=pl.CostEstimate(
            flops=40_000_000, bytes_accessed=70_000_000, transcendentals=0),
        scratch_types=[
            pltpu.VMEM((_B,), jnp.int32),
            pltpu.VMEM((_B + 2 * _L,), jnp.int32),
            pltpu.VMEM((_B + 2 * _L,), jnp.int32),
            pltpu.VMEM((_G, _D), jnp.float32),
            pltpu.VMEM((_G, _D), jnp.float32),
            pltpu.VMEM((_CPT + 1, _D), jnp.float32),
            pltpu.SMEM((_CPT + 1,), jnp.float32),
            pltpu.VMEM((_L,), jnp.float32),
            pltpu.SemaphoreType.DMA,
            pltpu.SemaphoreType.DMA,
        ],
    )(features, labels)
    ssq = pl.pallas_call(
        _ssq_body,
        grid=(_B // _RB,),
        in_specs=[pl.BlockSpec((_RB, _D), lambda i: (i, 0))],
        out_specs=pl.BlockSpec((1, 1), lambda i: (0, 0)),
        out_shape=jax.ShapeDtypeStruct((1, 1), jnp.float32),
        compiler_params=pltpu.CompilerParams(
            dimension_semantics=("arbitrary",)),
    )(features)
    return (ssq[0, 0] - jnp.sum(ctr_parts)) / (2.0 * features.shape[0])


def kernel(features, labels):
    return _center_loss_sc(features, labels)


# speedup vs baseline: 1.0615x; 1.0615x over previous
import jax
import jax.numpy as jnp
from jax import lax
from jax.experimental import pallas as pl
from jax.experimental.pallas import tpu as pltpu
from jax.experimental.pallas import tpu_sc as plsc

_NCLASS = 1000
_D = 512
_B = 16384
_NC = 2
_NS = 16
_NT = _NC * _NS
_L = 16
_CPT = 32
_G = 16
_NVEC = _D // _L
_RB = 2048


def _sc_body(feat_hbm, lab_hbm, out_hbm,
             labs_v, rowbuf, labbuf, rows_v, rows_w, tab_v, cnt_v,
             out_v, sem0, sem1):
    c = lax.axis_index("c")
    s = lax.axis_index("s")
    t = s * _NC + c
    zvec = jnp.zeros((_L,), jnp.float32)
    ovec = jnp.ones((_L,), jnp.float32)
    iota = lax.iota(jnp.int32, _L)
    five = jnp.full((_L,), 5, jnp.int32)
    onei = jnp.full((_L,), 1, jnp.int32)
    tvec = jnp.broadcast_to(t, (_L,))
    cvec0 = jnp.broadcast_to(t * _CPT, (_L,))
    dumpvec = jnp.broadcast_to(t * _CPT + _CPT, (_L,))

    pltpu.sync_copy(lab_hbm, labs_v)

    def zrow(i, _):
        for cc in range(_NVEC):
            tab_v[i, pl.ds(cc * _L, _L)] = zvec
        cnt_v[i] = 0.0
        return 0

    lax.fori_loop(0, _CPT + 1, zrow, 0)

    sixteen = jnp.full((_L,), _L, jnp.int32)
    _W = 4

    def scan_body(i, offv):
        lvs, ms, css, rids = [], [], [], []
        for k in range(_W):
            lv = labs_v[pl.ds((i * _W + k) * _L, _L)]
            m = lax.shift_right_logical(lv, five) == tvec
            lvs.append(lv)
            ms.append(m)
            css.append(plsc.cumsum(jnp.where(m, onei, onei - onei)))
            rids.append(iota + jnp.broadcast_to((i * _W + k) * _L, (_L,)))
        for k in range(_W):
            pos = (css[k] - onei) + offv
            plsc.store_scatter(rowbuf, [pos], rids[k], mask=ms[k])
            plsc.store_scatter(labbuf, [pos], lvs[k], mask=ms[k])
            offv = offv + plsc.all_reduce_population_count(ms[k])
        return offv

    offv = lax.fori_loop(0, _B // (_W * _L), scan_body,
                         jnp.zeros((_L,), jnp.int32))
    n = offv[0]
    zveci = jnp.zeros((_L,), jnp.int32)
    rowbuf[pl.ds(n, _L)] = zveci
    rowbuf[pl.ds(n + _L, _L)] = zveci
    labbuf[pl.ds(n, _L)] = dumpvec
    labbuf[pl.ds(n + _L, _L)] = dumpvec

    npair = (n + 2 * _G - 1) // (2 * _G)
    ntot2 = npair * 2

    def start(cid, buf, sem):
        pltpu.async_copy(feat_hbm.at[rowbuf.at[pl.ds(cid * _G, _G)]], buf, sem)

    def wait(cid, buf, sem):
        pltpu.make_async_copy(
            feat_hbm.at[rowbuf.at[pl.ds(cid * _G, _G)]], buf, sem).wait()

    @pl.when(ntot2 > 0)
    def _p0():
        start(0, rows_v, sem0)

    @pl.when(ntot2 > 1)
    def _p1():
        start(1, rows_w, sem1)

    def acc_chunk(cid, buf):
        lvec = labbuf[pl.ds(cid * _G, _L)] - cvec0
        for r in range(_G):
            lc = lvec[r]
            cnt_v[lc] = cnt_v[lc] + 1.0

            @plsc.parallel_loop(0, _NVEC, unroll=8)
            def _cc(cc):
                v = buf[r, pl.ds(cc * _L, _L)]
                plsc.addupdate(tab_v.at[lc, pl.ds(cc * _L, _L)], v)

    def pair(pid, _):
        cid0 = 2 * pid
        wait(cid0, rows_v, sem0)
        acc_chunk(cid0, rows_v)

        @pl.when(cid0 + 2 < ntot2)
        def _n0():
            start(cid0 + 2, rows_v, sem0)

        cid1 = cid0 + 1
        wait(cid1, rows_w, sem1)
        acc_chunk(cid1, rows_w)

        @pl.when(cid1 + 2 < ntot2)
        def _n1():
            start(cid1 + 2, rows_w, sem1)

        return 0

    lax.fori_loop(0, npair, pair, 0)

    def crow(lc, ctr):
        cvec = jnp.broadcast_to(cnt_v[lc], (_L,))
        inv = ovec / jnp.maximum(cvec, ovec)
        rowacc = zvec
        for cc in range(_NVEC):
            v = tab_v[lc, pl.ds(cc * _L, _L)]
            rowacc = rowacc + v * v
        return ctr + rowacc * inv

    ctr = lax.fori_loop(0, _CPT, crow, zvec)
    out_v[pl.ds(0, _L)] = ctr
    pltpu.sync_copy(out_v, out_hbm.at[t])


def _ssq_body(x_ref, o_ref):
    i = pl.program_id(0)

    @pl.when(i == 0)
    def _init():
        o_ref[...] = jnp.zeros_like(o_ref)

    x = x_ref[...]
    o_ref[...] = o_ref[...] + jnp.sum(x * x)


@jax.jit
def _center_loss_sc(features, labels):
    ctr_parts = pl.kernel(
        _sc_body,
        out_type=jax.ShapeDtypeStruct((_NT, _L), jnp.float32),
        mesh=plsc.VectorSubcoreMesh(core_axis_name="c", subcore_axis_name="s"),
        compiler_params=pltpu.CompilerParams(needs_layout_passes=False),
        cost_estimate=pl.CostEstimate(
            flops=40_000_000, bytes_accessed=70_000_000, transcendentals=0),
        scratch_types=[
            pltpu.VMEM((_B,), jnp.int32),
            pltpu.VMEM((_B + 2 * _L,), jnp.int32),
            pltpu.VMEM((_B + 2 * _L,), jnp.int32),
            pltpu.VMEM((_G, _D), jnp.float32),
            pltpu.VMEM((_G, _D), jnp.float32),
            pltpu.VMEM((_CPT + 1, _D), jnp.float32),
            pltpu.SMEM((_CPT + 1,), jnp.float32),
            pltpu.VMEM((_L,), jnp.float32),
            pltpu.SemaphoreType.DMA,
            pltpu.SemaphoreType.DMA,
        ],
    )(features, labels)
    ssq = pl.pallas_call(
        _ssq_body,
        grid=(_B // _RB,),
        in_specs=[pl.BlockSpec((_RB, _D), lambda i: (i, 0))],
        out_specs=pl.BlockSpec((1, 1), lambda i: (0, 0)),
        out_shape=jax.ShapeDtypeStruct((1, 1), jnp.float32),
        compiler_params=pltpu.CompilerParams(
            dimension_semantics=("arbitrary",)),
    )(features)
    return (ssq[0, 0] - jnp.sum(ctr_parts)) / (2.0 * features.shape[0])


def kernel(features, labels):
    return _center_loss_sc(features, labels)


# speedup vs baseline: 1.0755x; 1.0132x over previous
import jax
import jax.numpy as jnp
from jax import lax
from jax.experimental import pallas as pl
from jax.experimental.pallas import tpu as pltpu
from jax.experimental.pallas import tpu_sc as plsc

_NCLASS = 1000
_D = 512
_B = 16384
_NC = 2
_NS = 16
_NT = _NC * _NS
_L = 16
_CPT = 32
_G = 16
_NVEC = _D // _L
_RB = 2048


def _sc_body(feat_hbm, lab_hbm, out_hbm,
             labs_v, rowbuf, labbuf, rows_v, rows_w, tab_v, cnt_v,
             out_v, sem0, sem1):
    c = lax.axis_index("c")
    s = lax.axis_index("s")
    t = s * _NC + c
    zvec = jnp.zeros((_L,), jnp.float32)
    ovec = jnp.ones((_L,), jnp.float32)
    iota = lax.iota(jnp.int32, _L)
    five = jnp.full((_L,), 5, jnp.int32)
    onei = jnp.full((_L,), 1, jnp.int32)
    tvec = jnp.broadcast_to(t, (_L,))
    cvec0 = jnp.broadcast_to(t * _CPT, (_L,))
    dumpvec = jnp.broadcast_to(t * _CPT + _CPT, (_L,))

    pltpu.sync_copy(lab_hbm, labs_v)

    def zrow(i, _):
        for cc in range(_NVEC):
            tab_v[i, pl.ds(cc * _L, _L)] = zvec
        cnt_v[i] = 0.0
        return 0

    lax.fori_loop(0, _CPT + 1, zrow, 0)

    sixteen = jnp.full((_L,), _L, jnp.int32)
    _W = 8

    def scan_body(i, offv):
        lvs, ms, css, rids = [], [], [], []
        for k in range(_W):
            lv = labs_v[pl.ds((i * _W + k) * _L, _L)]
            m = lax.shift_right_logical(lv, five) == tvec
            lvs.append(lv)
            ms.append(m)
            css.append(plsc.cumsum(jnp.where(m, onei, onei - onei)))
            rids.append(iota + jnp.broadcast_to((i * _W + k) * _L, (_L,)))
        for k in range(_W):
            pos = (css[k] - onei) + offv
            plsc.store_scatter(rowbuf, [pos], rids[k], mask=ms[k])
            plsc.store_scatter(labbuf, [pos], lvs[k], mask=ms[k])
            offv = offv + plsc.all_reduce_population_count(ms[k])
        return offv

    offv = lax.fori_loop(0, _B // (_W * _L), scan_body,
                         jnp.zeros((_L,), jnp.int32))
    n = offv[0]
    zveci = jnp.zeros((_L,), jnp.int32)
    rowbuf[pl.ds(n, _L)] = zveci
    rowbuf[pl.ds(n + _L, _L)] = zveci
    labbuf[pl.ds(n, _L)] = dumpvec
    labbuf[pl.ds(n + _L, _L)] = dumpvec

    npair = (n + 2 * _G - 1) // (2 * _G)
    ntot2 = npair * 2

    def start(cid, buf, sem):
        pltpu.async_copy(feat_hbm.at[rowbuf.at[pl.ds(cid * _G, _G)]], buf, sem)

    def wait(cid, buf, sem):
        pltpu.make_async_copy(
            feat_hbm.at[rowbuf.at[pl.ds(cid * _G, _G)]], buf, sem).wait()

    @pl.when(ntot2 > 0)
    def _p0():
        start(0, rows_v, sem0)

    @pl.when(ntot2 > 1)
    def _p1():
        start(1, rows_w, sem1)

    def acc_chunk(cid, buf):
        lvec = labbuf[pl.ds(cid * _G, _L)] - cvec0
        for r in range(_G):
            lc = lvec[r]
            cnt_v[lc] = cnt_v[lc] + 1.0

            @plsc.parallel_loop(0, _NVEC, unroll=8)
            def _cc(cc):
                v = buf[r, pl.ds(cc * _L, _L)]
                plsc.addupdate(tab_v.at[lc, pl.ds(cc * _L, _L)], v)

    def pair(pid, _):
        cid0 = 2 * pid
        wait(cid0, rows_v, sem0)
        acc_chunk(cid0, rows_v)

        @pl.when(cid0 + 2 < ntot2)
        def _n0():
            start(cid0 + 2, rows_v, sem0)

        cid1 = cid0 + 1
        wait(cid1, rows_w, sem1)
        acc_chunk(cid1, rows_w)

        @pl.when(cid1 + 2 < ntot2)
        def _n1():
            start(cid1 + 2, rows_w, sem1)

        return 0

    lax.fori_loop(0, npair, pair, 0)

    def crow(lc, ctr):
        cvec = jnp.broadcast_to(cnt_v[lc], (_L,))
        inv = ovec / jnp.maximum(cvec, ovec)
        rowacc = zvec
        for cc in range(_NVEC):
            v = tab_v[lc, pl.ds(cc * _L, _L)]
            rowacc = rowacc + v * v
        return ctr + rowacc * inv

    ctr = lax.fori_loop(0, _CPT, crow, zvec)
    out_v[pl.ds(0, _L)] = ctr
    pltpu.sync_copy(out_v, out_hbm.at[t])


def _ssq_body(x_ref, o_ref):
    i = pl.program_id(0)

    @pl.when(i == 0)
    def _init():
        o_ref[...] = jnp.zeros_like(o_ref)

    x = x_ref[...]
    o_ref[...] = o_ref[...] + jnp.sum(x * x)


@jax.jit
def _center_loss_sc(features, labels):
    ctr_parts = pl.kernel(
        _sc_body,
        out_type=jax.ShapeDtypeStruct((_NT, _L), jnp.float32),
        mesh=plsc.VectorSubcoreMesh(core_axis_name="c", subcore_axis_name="s"),
        compiler_params=pltpu.CompilerParams(needs_layout_passes=False),
        cost_estimate=pl.CostEstimate(
            flops=40_000_000, bytes_accessed=70_000_000, transcendentals=0),
        scratch_types=[
            pltpu.VMEM((_B,), jnp.int32),
            pltpu.VMEM((_B + 2 * _L,), jnp.int32),
            pltpu.VMEM((_B + 2 * _L,), jnp.int32),
            pltpu.VMEM((_G, _D), jnp.float32),
            pltpu.VMEM((_G, _D), jnp.float32),
            pltpu.VMEM((_CPT + 1, _D), jnp.float32),
            pltpu.SMEM((_CPT + 1,), jnp.float32),
            pltpu.VMEM((_L,), jnp.float32),
            pltpu.SemaphoreType.DMA,
            pltpu.SemaphoreType.DMA,
        ],
    )(features, labels)
    ssq = pl.pallas_call(
        _ssq_body,
        grid=(_B // _RB,),
        in_specs=[pl.BlockSpec((_RB, _D), lambda i: (i, 0))],
        out_specs=pl.BlockSpec((1, 1), lambda i: (0, 0)),
        out_shape=jax.ShapeDtypeStruct((1, 1), jnp.float32),
        compiler_params=pltpu.CompilerParams(
            dimension_semantics=("arbitrary",)),
    )(features)
    return (ssq[0, 0] - jnp.sum(ctr_parts)) / (2.0 * features.shape[0])


def kernel(features, labels):
    return _center_loss_sc(features, labels)


# speedup vs baseline: 1.0863x; 1.0100x over previous
import jax
import jax.numpy as jnp
from jax import lax
from jax.experimental import pallas as pl
from jax.experimental.pallas import tpu as pltpu
from jax.experimental.pallas import tpu_sc as plsc

_NCLASS = 1000
_D = 512
_B = 16384
_NC = 2
_NS = 16
_NT = _NC * _NS
_L = 16
_CPT = 32
_G = 16
_NVEC = _D // _L
_RB = 2048


def _sc_body(feat_hbm, lab_hbm, out_hbm,
             labs_v, rowbuf, labbuf, rows_v, rows_w, tab_v, cnt_v,
             out_v, sem0, sem1):
    c = lax.axis_index("c")
    s = lax.axis_index("s")
    t = s * _NC + c
    zvec = jnp.zeros((_L,), jnp.float32)
    ovec = jnp.ones((_L,), jnp.float32)
    iota = lax.iota(jnp.int32, _L)
    five = jnp.full((_L,), 5, jnp.int32)
    onei = jnp.full((_L,), 1, jnp.int32)
    tvec = jnp.broadcast_to(t, (_L,))
    cvec0 = jnp.broadcast_to(t * _CPT, (_L,))
    dumpvec = jnp.broadcast_to(t * _CPT + _CPT, (_L,))

    lab_cp = pltpu.async_copy(lab_hbm, labs_v, sem0)

    def zrow(i, _):
        for cc in range(_NVEC):
            tab_v[i, pl.ds(cc * _L, _L)] = zvec
        cnt_v[i] = 0.0
        return 0

    lax.fori_loop(0, _CPT + 1, zrow, 0)
    lab_cp.wait()

    sixteen = jnp.full((_L,), _L, jnp.int32)
    _W = 8

    def scan_body(i, offv):
        lvs, ms, css, rids = [], [], [], []
        for k in range(_W):
            lv = labs_v[pl.ds((i * _W + k) * _L, _L)]
            m = lax.shift_right_logical(lv, five) == tvec
            lvs.append(lv)
            ms.append(m)
            css.append(plsc.cumsum(jnp.where(m, onei, onei - onei)))
            rids.append(iota + jnp.broadcast_to((i * _W + k) * _L, (_L,)))
        for k in range(_W):
            pos = (css[k] - onei) + offv
            plsc.store_scatter(rowbuf, [pos], rids[k], mask=ms[k])
            plsc.store_scatter(labbuf, [pos], lvs[k], mask=ms[k])
            offv = offv + plsc.all_reduce_population_count(ms[k])
        return offv

    offv = lax.fori_loop(0, _B // (_W * _L), scan_body,
                         jnp.zeros((_L,), jnp.int32))
    n = offv[0]
    zveci = jnp.zeros((_L,), jnp.int32)
    rowbuf[pl.ds(n, _L)] = zveci
    rowbuf[pl.ds(n + _L, _L)] = zveci
    labbuf[pl.ds(n, _L)] = dumpvec
    labbuf[pl.ds(n + _L, _L)] = dumpvec

    npair = (n + 2 * _G - 1) // (2 * _G)
    ntot2 = npair * 2

    def start(cid, buf, sem):
        pltpu.async_copy(feat_hbm.at[rowbuf.at[pl.ds(cid * _G, _G)]], buf, sem)

    def wait(cid, buf, sem):
        pltpu.make_async_copy(
            feat_hbm.at[rowbuf.at[pl.ds(cid * _G, _G)]], buf, sem).wait()

    @pl.when(ntot2 > 0)
    def _p0():
        start(0, rows_v, sem0)

    @pl.when(ntot2 > 1)
    def _p1():
        start(1, rows_w, sem1)

    def acc_chunk(cid, buf):
        lvec = labbuf[pl.ds(cid * _G, _L)] - cvec0
        for r in range(_G):
            lc = lvec[r]
            cnt_v[lc] = cnt_v[lc] + 1.0

            @plsc.parallel_loop(0, _NVEC, unroll=8)
            def _cc(cc):
                v = buf[r, pl.ds(cc * _L, _L)]
                plsc.addupdate(tab_v.at[lc, pl.ds(cc * _L, _L)], v)

    def pair(pid, _):
        cid0 = 2 * pid
        wait(cid0, rows_v, sem0)
        acc_chunk(cid0, rows_v)

        @pl.when(cid0 + 2 < ntot2)
        def _n0():
            start(cid0 + 2, rows_v, sem0)

        cid1 = cid0 + 1
        wait(cid1, rows_w, sem1)
        acc_chunk(cid1, rows_w)

        @pl.when(cid1 + 2 < ntot2)
        def _n1():
            start(cid1 + 2, rows_w, sem1)

        return 0

    lax.fori_loop(0, npair, pair, 0)

    def crow(lc, ctr):
        cvec = jnp.broadcast_to(cnt_v[lc], (_L,))
        inv = ovec / jnp.maximum(cvec, ovec)
        accs = [zvec, zvec, zvec, zvec]
        for cc in range(_NVEC):
            v = tab_v[lc, pl.ds(cc * _L, _L)]
            accs[cc % 4] = accs[cc % 4] + v * v
        rowacc = (accs[0] + accs[1]) + (accs[2] + accs[3])
        return ctr + rowacc * inv

    ctr = lax.fori_loop(0, _CPT, crow, zvec)
    out_v[pl.ds(0, _L)] = ctr
    pltpu.sync_copy(out_v, out_hbm.at[t])


def _ssq_body(x_ref, o_ref):
    i = pl.program_id(0)

    @pl.when(i == 0)
    def _init():
        o_ref[...] = jnp.zeros_like(o_ref)

    x = x_ref[...]
    o_ref[...] = o_ref[...] + jnp.sum(x * x)


@jax.jit
def _center_loss_sc(features, labels):
    ctr_parts = pl.kernel(
        _sc_body,
        out_type=jax.ShapeDtypeStruct((_NT, _L), jnp.float32),
        mesh=plsc.VectorSubcoreMesh(core_axis_name="c", subcore_axis_name="s"),
        compiler_params=pltpu.CompilerParams(needs_layout_passes=False),
        cost_estimate=pl.CostEstimate(
            flops=40_000_000, bytes_accessed=70_000_000, transcendentals=0),
        scratch_types=[
            pltpu.VMEM((_B,), jnp.int32),
            pltpu.VMEM((_B + 2 * _L,), jnp.int32),
            pltpu.VMEM((_B + 2 * _L,), jnp.int32),
            pltpu.VMEM((_G, _D), jnp.float32),
            pltpu.VMEM((_G, _D), jnp.float32),
            pltpu.VMEM((_CPT + 1, _D), jnp.float32),
            pltpu.SMEM((_CPT + 1,), jnp.float32),
            pltpu.VMEM((_L,), jnp.float32),
            pltpu.SemaphoreType.DMA,
            pltpu.SemaphoreType.DMA,
        ],
    )(features, labels)
    ssq = pl.pallas_call(
        _ssq_body,
        grid=(_B // _RB,),
        in_specs=[pl.BlockSpec((_RB, _D), lambda i: (i, 0))],
        out_specs=pl.BlockSpec((1, 1), lambda i: (0, 0)),
        out_shape=jax.ShapeDtypeStruct((1, 1), jnp.float32),
        compiler_params=pltpu.CompilerParams(
            dimension_semantics=("arbitrary",)),
    )(features)
    return (ssq[0, 0] - jnp.sum(ctr_parts)) / (2.0 * features.shape[0])


def kernel(features, labels):
    return _center_loss_sc(features, labels)
